# baseline (speedup 1.0000x reference)
import functools

import jax
import jax.numpy as jnp
from jax import lax
from jax.experimental import pallas as pl
from jax.experimental.pallas import tpu as pltpu
from jax.experimental.pallas import tpu_sc as plsc

N = 10000
E = 320000
IN = 128
HID = 256
OUT = 128
NEXP = 8
NG = 64
RH = 128
HALF = 128

NCORES = 2
NTILES = 16

W = 128
EP = 157 * NTILES * W
WPT = EP // (NTILES * W)
NPAD = N + 16
ZR = 632
ZR_LAST = NPAD - 15 * ZR
ORT = 632
ORT_LAST = N - 15 * ORT

RB = 2000
NBLK = N // RB
EB = 20000
NEB = E // EB


def _make_seg_body(ke, off_fn):
    NS = 3

    def body(x, src, dst, zrows, out, svs, dvs, rvs, gsems, ssems, acc):
        c = lax.axis_index("c")
        s = lax.axis_index("s")
        ebase = s * (WPT * W)

        def load_idx(w, sv, dv, roff):
            off = ebase + w * W
            pltpu.sync_copy(src.at[pl.ds(off, W)], sv)
            for j in range(W // 16):
                sv[pl.ds(j * 16, 16)] = sv[pl.ds(j * 16, 16)] + roff
            pltpu.sync_copy(dst.at[pl.ds(off, W)], dv)

        for e in range(ke):
            roff = off_fn(c, e)

            @pl.when(s < NTILES - 1)
            def _():
                pltpu.sync_copy(zrows, acc.at[pl.ds(s * ZR, ZR)])

            @pl.when(s == NTILES - 1)
            def _():
                pltpu.sync_copy(zrows.at[pl.ds(0, ZR_LAST)],
                                acc.at[pl.ds(15 * ZR, ZR_LAST)])

            plsc.subcore_barrier()

            for b in range(2):
                load_idx(b, svs[b], dvs[b], roff)
                pltpu.async_copy(x.at[svs[b]], rvs[b], gsems[b])

            def visit(w, carry):
                for b in range(NS):
                    @pl.when(w % NS == b)
                    def _(b=b):
                        pltpu.make_async_copy(x.at[svs[b]], rvs[b],
                                              gsems[b]).wait()
                        pltpu.async_copy(rvs[b], acc.at[dvs[b]], ssems[b],
                                         add=True)
                @pl.when(w + 2 < WPT)
                def _():
                    for b in range(NS):
                        @pl.when((w + 2) % NS == b)
                        def _(b=b):
                            @pl.when(w >= 1)
                            def _():
                                pltpu.make_async_copy(
                                    rvs[b], acc.at[dvs[b]], ssems[b]).wait()
                            load_idx(w + 2, svs[b], dvs[b], roff)
                            pltpu.async_copy(x.at[svs[b]], rvs[b], gsems[b])
                return carry

            lax.fori_loop(0, WPT, visit, 0)
            for wlast in range(WPT - 3, WPT):
                b = wlast % NS
                pltpu.make_async_copy(rvs[b], acc.at[dvs[b]], ssems[b]).wait()
            plsc.subcore_barrier()

            @pl.when(s < NTILES - 1)
            def _():
                pltpu.sync_copy(acc.at[pl.ds(s * ORT, ORT)],
                                out.at[pl.ds(roff + s * ORT, ORT)])

            @pl.when(s == NTILES - 1)
            def _():
                pltpu.sync_copy(acc.at[pl.ds(15 * ORT, ORT_LAST)],
                                out.at[pl.ds(roff + 15 * ORT, ORT_LAST)])

    return body


@functools.cache
def _seg_sc(n_rows, ke, kind):
    if kind == "halves":
        off_fn = lambda c, e: (e * NCORES + c) * N
    else:
        off_fn = lambda c, e: (c * ke + e) * N
    return pl.kernel(
        _make_seg_body(ke, off_fn),
        out_type=jax.ShapeDtypeStruct((n_rows, HALF), jnp.float32),
        mesh=plsc.VectorSubcoreMesh(core_axis_name="c", subcore_axis_name="s",
                                    num_cores=NCORES, num_subcores=NTILES),
        scratch_types=[
            [pltpu.VMEM((W,), jnp.int32) for _ in range(3)],
            [pltpu.VMEM((W,), jnp.int32) for _ in range(3)],
            [pltpu.VMEM((W, HALF), jnp.float32) for _ in range(3)],
            [pltpu.SemaphoreType.DMA for _ in range(3)],
            [pltpu.SemaphoreType.DMA for _ in range(3)],
            pltpu.VMEM_SHARED((NPAD, HALF), jnp.float32),
        ],
    )


def _seg1(xflat, srcp, dstp, zrows):
    return _seg_sc(2 * N, 1, "halves")(xflat, srcp, dstp, zrows)


def _seg4(xflat, srcp, dstp, zrows):
    return _seg_sc(8 * N, 4, "halves")(xflat, srcp, dstp, zrows)


def _segm2(xflat, srcp, dstp, zrows):
    return _seg_sc(4 * N, 2, "whole")(xflat, srcp, dstp, zrows)


def _enc_body(x_ref, w1t_ref, b1_ref, w2t_ref, b2_ref, out_ref):
    h1 = jnp.maximum(
        jnp.dot(x_ref[...], w1t_ref[...], preferred_element_type=jnp.float32)
        + b1_ref[...], 0.0)
    hh = jnp.dot(h1, w2t_ref[...], preferred_element_type=jnp.float32) + b2_ref[...]
    out_ref[0] = hh[:, :HALF]
    out_ref[1] = hh[:, HALF:]


def _enc(x, w1t, b1r, w2t, b2r):
    return pl.pallas_call(
        _enc_body,
        grid=(NBLK,),
        in_specs=[
            pl.BlockSpec((RB, IN), lambda i: (i, 0)),
            pl.BlockSpec((IN, HID), lambda i: (0, 0)),
            pl.BlockSpec((1, HID), lambda i: (0, 0)),
            pl.BlockSpec((HID, HID), lambda i: (0, 0)),
            pl.BlockSpec((1, HID), lambda i: (0, 0)),
        ],
        out_specs=pl.BlockSpec((2, RB, HALF), lambda i: (0, i, 0)),
        out_shape=jax.ShapeDtypeStruct((2, N, HALF), jnp.float32),
    )(x, w1t, b1r, w2t, b2r)


def _stats_body(batch_ref, src_ref, nraw_ref, f_ref, starts_ref):
    i = pl.program_id(0)

    @pl.when(i == 0)
    def _():
        b = batch_ref[...]
        g = lax.broadcasted_iota(jnp.int32, (NG, 1), 0)
        nr = jnp.sum((b == g).astype(jnp.float32), axis=1, keepdims=True)
        nraw_ref[...] = nr
        lt = (lax.broadcasted_iota(jnp.int32, (NG + 1, NG), 1)
              < lax.broadcasted_iota(jnp.int32, (NG + 1, NG), 0)).astype(jnp.float32)
        starts_ref[...] = jnp.dot(lt, nr, preferred_element_type=jnp.float32)
        f_ref[...] = jnp.zeros((NG + 1, 1), jnp.float32)

    @pl.when(i > 0)
    def _():
        s = src_ref[0].astype(jnp.float32)
        st = starts_ref[...]
        f_ref[...] += jnp.sum((s < st).astype(jnp.float32), axis=1, keepdims=True)


def _stats(batch_row, src3):
    return pl.pallas_call(
        _stats_body,
        grid=(NEB + 1,),
        in_specs=[
            pl.BlockSpec((1, N), lambda i: (0, 0)),
            pl.BlockSpec((1, 1, EB), lambda i: (jnp.maximum(i - 1, 0), 0, 0)),
        ],
        out_specs=[
            pl.BlockSpec((NG, 1), lambda i: (0, 0)),
            pl.BlockSpec((NG + 1, 1), lambda i: (0, 0)),
        ],
        out_shape=[
            jax.ShapeDtypeStruct((NG, 1), jnp.float32),
            jax.ShapeDtypeStruct((NG + 1, 1), jnp.float32),
        ],
        scratch_shapes=[pltpu.VMEM((NG + 1, 1), jnp.float32)],
    )(batch_row, src3)


def _pergraph_body(nraw_ref, f_ref, gfn_ref, lnn_ref):
    nr = nraw_ref[...]
    f = f_ref[...]
    e = f[1:NG + 1] - f[0:NG]
    n = jnp.maximum(nr, 1.0)
    dens = e / jnp.maximum(n * (n - 1.0), 1.0)
    ln = jnp.log(n)
    lnn = (ln - jnp.min(ln)) / (jnp.max(ln) - jnp.min(ln) + 1e-06)
    gf = jnp.concatenate([n, e, dens], axis=1)
    mu = jnp.mean(gf, axis=0, keepdims=True)
    sd = jnp.sqrt(jnp.mean((gf - mu) ** 2, axis=0, keepdims=True))
    gfn_ref[...] = (gf - mu) / (sd + 1e-06)
    lnn_ref[...] = lnn


def _pergraph(nraw, f):
    return pl.pallas_call(
        _pergraph_body,
        out_shape=[
            jax.ShapeDtypeStruct((NG, 3), jnp.float32),
            jax.ShapeDtypeStruct((NG, 1), jnp.float32),
        ],
    )(nraw, f)


def _router_body(h_ref, batch_ref, gfnt_ref, lnn_ref, cen_ref, w1t_ref, b1_ref,
                 w2t_ref, b2_ref, out_ref):
    h = jnp.concatenate([h_ref[0], h_ref[1]], axis=1)
    g = lax.broadcasted_iota(jnp.int32, (1, NG), 1)
    oh = batch_ref[...] == g
    def sel(row):
        return jnp.sum(jnp.where(oh, row, 0.0), axis=1, keepdims=True)
    sf = jnp.concatenate([sel(gfnt_ref[0:1]), sel(gfnt_ref[1:2]),
                          sel(gfnt_ref[2:3])], axis=1)
    lnn = sel(lnn_ref[...])
    rin = jnp.concatenate([h, sf], axis=1)
    pre = jnp.dot(rin, w1t_ref[...], preferred_element_type=jnp.float32) + b1_ref[...]
    a = jnp.maximum(pre, 0.0)
    logits = jnp.dot(a, w2t_ref[...], preferred_element_type=jnp.float32) + b2_ref[...]
    prior = -(lnn - cen_ref[...]) ** 2
    logits = (1.0 - 0.35) * logits + 0.35 * prior
    m = jnp.max(logits, axis=1, keepdims=True)
    ex = jnp.exp(logits - m)
    t4 = ex[:, 0:4] + ex[:, 4:8]
    t2 = t4[:, 0:2] + t4[:, 2:4]
    probs = ex / (t2[:, 0:1] + t2[:, 1:2])
    i8 = lax.broadcasted_iota(jnp.int32, (1, NEXP), 1)
    m1 = jnp.max(probs, axis=1, keepdims=True)
    idx1 = jnp.min(jnp.where(probs == m1, i8, 99), axis=1, keepdims=True)
    mask1 = (i8 == idx1)
    pno1 = jnp.where(mask1, -1.0, probs)
    m2 = jnp.max(pno1, axis=1, keepdims=True)
    idx2 = jnp.min(jnp.where(pno1 == m2, i8, 99), axis=1, keepdims=True)
    mask2 = (i8 == idx2)
    denom = m1 + m2 + 1e-08
    out_ref[...] = (mask1 * (m1 / denom) + mask2 * (m2 / denom)).astype(jnp.float32)


def _router(hcat, batch_col, gfnt, lnn_row, cen, w1t, b1r, w2t, b2r):
    return pl.pallas_call(
        _router_body,
        grid=(NBLK,),
        in_specs=[
            pl.BlockSpec((2, RB, HALF), lambda i: (0, i, 0)),
            pl.BlockSpec((RB, 1), lambda i: (i, 0)),
            pl.BlockSpec((3, NG), lambda i: (0, 0)),
            pl.BlockSpec((1, NG), lambda i: (0, 0)),
            pl.BlockSpec((1, NEXP), lambda i: (0, 0)),
            pl.BlockSpec((HID + 3, RH), lambda i: (0, 0)),
            pl.BlockSpec((1, RH), lambda i: (0, 0)),
            pl.BlockSpec((RH, NEXP), lambda i: (0, 0)),
            pl.BlockSpec((1, NEXP), lambda i: (0, 0)),
        ],
        out_specs=pl.BlockSpec((RB, NEXP), lambda i: (i, 0)),
        out_shape=jax.ShapeDtypeStruct((N, NEXP), jnp.float32),
    )(hcat, batch_col, gfnt, lnn_row, cen, w1t, b1r, w2t, b2r)


def _gconv_body(agg_ref, x_ref, wrel_ref, wroot_ref, b_ref, out_ref):
    agg = jnp.concatenate([agg_ref[0, 0], agg_ref[0, 1]], axis=1)
    xx = jnp.concatenate([x_ref[0, 0], x_ref[0, 1]], axis=1)
    dn = (((1,), (1,)), ((), ()))
    z = (lax.dot_general(agg, wrel_ref[0], dn, preferred_element_type=jnp.float32)
         + lax.dot_general(xx, wroot_ref[0], dn, preferred_element_type=jnp.float32)
         + b_ref[0])
    z = jnp.maximum(z, 0.0)
    out_ref[0, 0] = z[:, :HALF]
    out_ref[0, 1] = z[:, HALF:]


def _gconv(agg4, x4, wrel, wroot, b3d):
    ka = agg4.shape[0]
    kx = x4.shape[0]
    return pl.pallas_call(
        _gconv_body,
        grid=(NEXP, NBLK),
        in_specs=[
            pl.BlockSpec((1, 2, RB, HALF),
                         lambda e, i, ka=ka: (jnp.minimum(e, ka - 1), 0, i, 0)),
            pl.BlockSpec((1, 2, RB, HALF),
                         lambda e, i, kx=kx: (jnp.minimum(e, kx - 1), 0, i, 0)),
            pl.BlockSpec((1, HID, HID), lambda e, i: (e, 0, 0)),
            pl.BlockSpec((1, HID, HID), lambda e, i: (e, 0, 0)),
            pl.BlockSpec((1, 1, HID), lambda e, i: (e, 0, 0)),
        ],
        out_specs=pl.BlockSpec((1, 2, RB, HALF), lambda e, i: (e, 0, i, 0)),
        out_shape=jax.ShapeDtypeStruct((NEXP, 2, N, HALF), jnp.float32),
    )(agg4, x4, wrel, wroot, b3d)


def _gconv2_body(agg_ref, x_ref, wrel_ref, wroot_ref, b_ref, wrel3_ref,
                 out_ref, m_ref):
    agg = jnp.concatenate([agg_ref[0, 0], agg_ref[0, 1]], axis=1)
    xx = jnp.concatenate([x_ref[0, 0], x_ref[0, 1]], axis=1)
    dn = (((1,), (1,)), ((), ()))
    z = (lax.dot_general(agg, wrel_ref[0], dn, preferred_element_type=jnp.float32)
         + lax.dot_general(xx, wroot_ref[0], dn, preferred_element_type=jnp.float32)
         + b_ref[0])
    z = jnp.maximum(z, 0.0)
    out_ref[0, 0] = z[:, :HALF]
    out_ref[0, 1] = z[:, HALF:]
    m_ref[0] = lax.dot_general(z, wrel3_ref[0], dn,
                               preferred_element_type=jnp.float32)


def _gconv2(agg4, x4, wrel, wroot, b3d, wrel3):
    ne = wrel.shape[0]
    return pl.pallas_call(
        _gconv2_body,
        grid=(ne, NBLK),
        in_specs=[
            pl.BlockSpec((1, 2, RB, HALF), lambda e, i: (e, 0, i, 0)),
            pl.BlockSpec((1, 2, RB, HALF), lambda e, i: (e, 0, i, 0)),
            pl.BlockSpec((1, HID, HID), lambda e, i: (e, 0, 0)),
            pl.BlockSpec((1, HID, HID), lambda e, i: (e, 0, 0)),
            pl.BlockSpec((1, 1, HID), lambda e, i: (e, 0, 0)),
            pl.BlockSpec((1, OUT, HID), lambda e, i: (e, 0, 0)),
        ],
        out_specs=[
            pl.BlockSpec((1, 2, RB, HALF), lambda e, i: (e, 0, i, 0)),
            pl.BlockSpec((1, RB, OUT), lambda e, i: (e, i, 0)),
        ],
        out_shape=[
            jax.ShapeDtypeStruct((ne, 2, N, HALF), jnp.float32),
            jax.ShapeDtypeStruct((ne, N, OUT), jnp.float32),
        ],
    )(agg4, x4, wrel, wroot, b3d, wrel3)


def _make_l3_body(ebase, with_init):
    def body(*refs):
        if with_init:
            agg_ref, x_ref, wroot_ref, b_ref, w_ref, init_ref, out_ref = refs
        else:
            agg_ref, x_ref, wroot_ref, b_ref, w_ref = refs[:5]
            out_ref = refs[5]
        e = pl.program_id(1)
        xx = jnp.concatenate([x_ref[0, 0], x_ref[0, 1]], axis=1)
        dn = (((1,), (1,)), ((), ()))
        o = (agg_ref[0]
             + lax.dot_general(xx, wroot_ref[0], dn,
                               preferred_element_type=jnp.float32)
             + b_ref[0])
        i8 = lax.broadcasted_iota(jnp.int32, (1, NEXP), 1)
        wcol = jnp.sum(w_ref[...] * (i8 == e + ebase).astype(jnp.float32),
                       axis=1, keepdims=True)

        @pl.when(e == 0)
        def _():
            if with_init:
                out_ref[...] = init_ref[...] + o * wcol
            else:
                out_ref[...] = o * wcol

        @pl.when(e > 0)
        def _():
            out_ref[...] += o * wcol

    return body


def _l3(agg3, x4, wroot, b3d, wts, ebase, init=None):
    ne = wroot.shape[0]
    in_specs = [
        pl.BlockSpec((1, RB, OUT), lambda i, e: (e, i, 0)),
        pl.BlockSpec((1, 2, RB, HALF), lambda i, e: (e, 0, i, 0)),
        pl.BlockSpec((1, OUT, HID), lambda i, e: (e, 0, 0)),
        pl.BlockSpec((1, 1, OUT), lambda i, e: (e, 0, 0)),
        pl.BlockSpec((RB, NEXP), lambda i, e: (i, 0)),
    ]
    args = [agg3, x4, wroot, b3d, wts]
    if init is not None:
        in_specs.append(pl.BlockSpec((RB, OUT), lambda i, e: (i, 0)))
        args.append(init)
    return pl.pallas_call(
        _make_l3_body(ebase, init is not None),
        grid=(NBLK, ne),
        in_specs=in_specs,
        out_specs=pl.BlockSpec((RB, OUT), lambda i, e: (i, 0)),
        out_shape=jax.ShapeDtypeStruct((N, OUT), jnp.float32),
    )(*args)


def kernel(x, edge_index, batch, W_enc1, b_enc1, W_enc2, b_enc2, W_r1, b_r1,
           W_r2, b_r2, size_centers, Wrel1, Wroot1, b1, Wrel2, Wroot2, b2,
           Wrel3, Wroot3, b3):
    src = edge_index[0]
    dst = edge_index[1]

    pad = EP - E
    padrows = (jnp.arange(pad, dtype=jnp.int32) % 16)
    srcp = jnp.concatenate([src, padrows])
    dstp = jnp.concatenate([dst, N + padrows])
    zrows = jnp.zeros((ZR, HALF), jnp.float32)

    batch_row = batch.reshape(1, N)
    batch_col = batch.reshape(N, 1)
    src3 = src.reshape(NEB, 1, EB)

    w1t = W_enc1.T
    b1r = b_enc1.reshape(1, HID)
    w2t = W_enc2.T
    b2r = b_enc2.reshape(1, HID)
    w1rt = W_r1.T
    br1 = b_r1.reshape(1, RH)
    w2rt = W_r2.T
    br2 = b_r2.reshape(1, NEXP)
    cen = size_centers.reshape(1, NEXP)
    b1_3 = b1.reshape(NEXP, 1, HID)
    b2_3 = b2.reshape(NEXP, 1, HID)
    b3_3 = b3.reshape(NEXP, 1, OUT)

    hcat = _enc(x, w1t, b1r, w2t, b2r)
    nraw, f = _stats(batch_row, src3)
    gfn, lnn_g = _pergraph(nraw, f)
    wts = _router(hcat, batch_col, gfn.T, lnn_g.T, cen, w1rt, br1, w2rt, br2)

    agg_h = _seg1(hcat.reshape(2 * N, HALF), srcp, dstp, zrows)
    h1 = _gconv(agg_h.reshape(1, 2, N, HALF), hcat.reshape(1, 2, N, HALF),
                Wrel1, Wroot1, b1_3)

    G = NEXP // 2
    h1a, h1b = h1[:G], h1[G:]
    agg1a = _seg4(h1a.reshape(2 * G * N, HALF), srcp, dstp,
                  zrows).reshape(G, 2, N, HALF)
    h2a, m2a = _gconv2(agg1a, h1a, Wrel2[:G], Wroot2[:G], b2_3[:G], Wrel3[:G])
    agg1b = _seg4(h1b.reshape(2 * G * N, HALF), srcp, dstp,
                  zrows).reshape(G, 2, N, HALF)
    agg3a = _segm2(m2a.reshape(G * N, HALF), srcp, dstp,
                   zrows).reshape(G, N, OUT)
    h2b, m2b = _gconv2(agg1b, h1b, Wrel2[G:], Wroot2[G:], b2_3[G:], Wrel3[G:])
    agg3b = _segm2(m2b.reshape(G * N, HALF), srcp, dstp,
                   zrows).reshape(G, N, OUT)
    pred_a = _l3(agg3a, h2a, Wroot3[:G], b3_3[:G], wts, 0)
    pred = _l3(agg3b, h2b, Wroot3[G:], b3_3[G:], wts, G, init=pred_a)
    return pred


# speedup vs baseline: 1.0056x; 1.0056x over previous
import functools

import jax
import jax.numpy as jnp
from jax import lax
from jax.experimental import pallas as pl
from jax.experimental.pallas import tpu as pltpu
from jax.experimental.pallas import tpu_sc as plsc

N = 10000
E = 320000
IN = 128
HID = 256
OUT = 128
NEXP = 8
NG = 64
RH = 128
HALF = 128

NCORES = 2
NTILES = 16

W = 128
EP = 157 * NTILES * W
WPT = EP // (NTILES * W)
NPAD = N + 16
ZR = 632
ZR_LAST = NPAD - 15 * ZR
ORT = 632
ORT_LAST = N - 15 * ORT

RB = 2000
NBLK = N // RB
EB = 20000
NEB = E // EB


def _make_seg_body(ke, off_fn):
    NS = 3

    def body(x, src, dst, zrows, out, svs, dvs, rvs, gsems, ssems, acc):
        c = lax.axis_index("c")
        s = lax.axis_index("s")
        ebase = s * (WPT * W)

        def load_idx(w, sv, dv, roff):
            off = ebase + w * W
            pltpu.sync_copy(src.at[pl.ds(off, W)], sv)
            for j in range(W // 16):
                sv[pl.ds(j * 16, 16)] = sv[pl.ds(j * 16, 16)] + roff
            pltpu.sync_copy(dst.at[pl.ds(off, W)], dv)

        for e in range(ke):
            roff = off_fn(c, e)

            @pl.when(s < NTILES - 1)
            def _():
                pltpu.sync_copy(zrows, acc.at[pl.ds(s * ZR, ZR)])

            @pl.when(s == NTILES - 1)
            def _():
                pltpu.sync_copy(zrows.at[pl.ds(0, ZR_LAST)],
                                acc.at[pl.ds(15 * ZR, ZR_LAST)])

            plsc.subcore_barrier()

            for b in range(2):
                load_idx(b, svs[b], dvs[b], roff)
                pltpu.async_copy(x.at[svs[b]], rvs[b], gsems[b])

            def visit(w, carry):
                for b in range(NS):
                    @pl.when(w % NS == b)
                    def _(b=b):
                        pltpu.make_async_copy(x.at[svs[b]], rvs[b],
                                              gsems[b]).wait()
                        pltpu.async_copy(rvs[b], acc.at[dvs[b]], ssems[b],
                                         add=True)
                @pl.when(w + 2 < WPT)
                def _():
                    for b in range(NS):
                        @pl.when((w + 2) % NS == b)
                        def _(b=b):
                            @pl.when(w >= 1)
                            def _():
                                pltpu.make_async_copy(
                                    rvs[b], acc.at[dvs[b]], ssems[b]).wait()
                            load_idx(w + 2, svs[b], dvs[b], roff)
                            pltpu.async_copy(x.at[svs[b]], rvs[b], gsems[b])
                return carry

            lax.fori_loop(0, WPT, visit, 0)
            for wlast in range(WPT - 3, WPT):
                b = wlast % NS
                pltpu.make_async_copy(rvs[b], acc.at[dvs[b]], ssems[b]).wait()
            plsc.subcore_barrier()

            @pl.when(s < NTILES - 1)
            def _():
                pltpu.sync_copy(acc.at[pl.ds(s * ORT, ORT)],
                                out.at[pl.ds(roff + s * ORT, ORT)])

            @pl.when(s == NTILES - 1)
            def _():
                pltpu.sync_copy(acc.at[pl.ds(15 * ORT, ORT_LAST)],
                                out.at[pl.ds(roff + 15 * ORT, ORT_LAST)])

    return body


@functools.cache
def _seg_sc(n_rows, ke, kind):
    if kind == "halves":
        off_fn = lambda c, e: (e * NCORES + c) * N
    else:
        off_fn = lambda c, e: (c * ke + e) * N
    return pl.kernel(
        _make_seg_body(ke, off_fn),
        out_type=jax.ShapeDtypeStruct((n_rows, HALF), jnp.float32),
        mesh=plsc.VectorSubcoreMesh(core_axis_name="c", subcore_axis_name="s",
                                    num_cores=NCORES, num_subcores=NTILES),
        scratch_types=[
            [pltpu.VMEM((W,), jnp.int32) for _ in range(3)],
            [pltpu.VMEM((W,), jnp.int32) for _ in range(3)],
            [pltpu.VMEM((W, HALF), jnp.float32) for _ in range(3)],
            [pltpu.SemaphoreType.DMA for _ in range(3)],
            [pltpu.SemaphoreType.DMA for _ in range(3)],
            pltpu.VMEM_SHARED((NPAD, HALF), jnp.float32),
        ],
    )


def _seg1(xflat, srcp, dstp, zrows):
    return _seg_sc(2 * N, 1, "halves")(xflat, srcp, dstp, zrows)


def _seg8(xflat, srcp, dstp, zrows):
    return _seg_sc(2 * NEXP * N, NEXP, "halves")(xflat, srcp, dstp, zrows)


def _segm(xflat, srcp, dstp, zrows):
    return _seg_sc(NEXP * N, NEXP // NCORES, "whole")(xflat, srcp, dstp, zrows)


def _enc_body(x_ref, w1t_ref, b1_ref, w2t_ref, b2_ref, out_ref):
    h1 = jnp.maximum(
        jnp.dot(x_ref[...], w1t_ref[...], preferred_element_type=jnp.float32)
        + b1_ref[...], 0.0)
    hh = jnp.dot(h1, w2t_ref[...], preferred_element_type=jnp.float32) + b2_ref[...]
    out_ref[0] = hh[:, :HALF]
    out_ref[1] = hh[:, HALF:]


def _enc(x, w1t, b1r, w2t, b2r):
    return pl.pallas_call(
        _enc_body,
        grid=(NBLK,),
        in_specs=[
            pl.BlockSpec((RB, IN), lambda i: (i, 0)),
            pl.BlockSpec((IN, HID), lambda i: (0, 0)),
            pl.BlockSpec((1, HID), lambda i: (0, 0)),
            pl.BlockSpec((HID, HID), lambda i: (0, 0)),
            pl.BlockSpec((1, HID), lambda i: (0, 0)),
        ],
        out_specs=pl.BlockSpec((2, RB, HALF), lambda i: (0, i, 0)),
        out_shape=jax.ShapeDtypeStruct((2, N, HALF), jnp.float32),
    )(x, w1t, b1r, w2t, b2r)


def _stats_body(batch_ref, src_ref, nraw_ref, f_ref, starts_ref):
    i = pl.program_id(0)

    @pl.when(i == 0)
    def _():
        b = batch_ref[...]
        g = lax.broadcasted_iota(jnp.int32, (NG, 1), 0)
        nr = jnp.sum((b == g).astype(jnp.float32), axis=1, keepdims=True)
        nraw_ref[...] = nr
        lt = (lax.broadcasted_iota(jnp.int32, (NG + 1, NG), 1)
              < lax.broadcasted_iota(jnp.int32, (NG + 1, NG), 0)).astype(jnp.float32)
        starts_ref[...] = jnp.dot(lt, nr, preferred_element_type=jnp.float32)
        f_ref[...] = jnp.zeros((NG + 1, 1), jnp.float32)

    @pl.when(i > 0)
    def _():
        s = src_ref[0].astype(jnp.float32)
        st = starts_ref[...]
        f_ref[...] += jnp.sum((s < st).astype(jnp.float32), axis=1, keepdims=True)


def _stats(batch_row, src3):
    return pl.pallas_call(
        _stats_body,
        grid=(NEB + 1,),
        in_specs=[
            pl.BlockSpec((1, N), lambda i: (0, 0)),
            pl.BlockSpec((1, 1, EB), lambda i: (jnp.maximum(i - 1, 0), 0, 0)),
        ],
        out_specs=[
            pl.BlockSpec((NG, 1), lambda i: (0, 0)),
            pl.BlockSpec((NG + 1, 1), lambda i: (0, 0)),
        ],
        out_shape=[
            jax.ShapeDtypeStruct((NG, 1), jnp.float32),
            jax.ShapeDtypeStruct((NG + 1, 1), jnp.float32),
        ],
        scratch_shapes=[pltpu.VMEM((NG + 1, 1), jnp.float32)],
    )(batch_row, src3)


def _pergraph_body(nraw_ref, f_ref, gfn_ref, lnn_ref):
    nr = nraw_ref[...]
    f = f_ref[...]
    e = f[1:NG + 1] - f[0:NG]
    n = jnp.maximum(nr, 1.0)
    dens = e / jnp.maximum(n * (n - 1.0), 1.0)
    ln = jnp.log(n)
    lnn = (ln - jnp.min(ln)) / (jnp.max(ln) - jnp.min(ln) + 1e-06)
    gf = jnp.concatenate([n, e, dens], axis=1)
    mu = jnp.mean(gf, axis=0, keepdims=True)
    sd = jnp.sqrt(jnp.mean((gf - mu) ** 2, axis=0, keepdims=True))
    gfn_ref[...] = (gf - mu) / (sd + 1e-06)
    lnn_ref[...] = lnn


def _pergraph(nraw, f):
    return pl.pallas_call(
        _pergraph_body,
        out_shape=[
            jax.ShapeDtypeStruct((NG, 3), jnp.float32),
            jax.ShapeDtypeStruct((NG, 1), jnp.float32),
        ],
    )(nraw, f)


def _router_body(h_ref, batch_ref, gfnt_ref, lnn_ref, cen_ref, w1t_ref, b1_ref,
                 w2t_ref, b2_ref, out_ref):
    h = jnp.concatenate([h_ref[0], h_ref[1]], axis=1)
    g = lax.broadcasted_iota(jnp.int32, (1, NG), 1)
    oh = batch_ref[...] == g
    def sel(row):
        return jnp.sum(jnp.where(oh, row, 0.0), axis=1, keepdims=True)
    sf = jnp.concatenate([sel(gfnt_ref[0:1]), sel(gfnt_ref[1:2]),
                          sel(gfnt_ref[2:3])], axis=1)
    lnn = sel(lnn_ref[...])
    rin = jnp.concatenate([h, sf], axis=1)
    pre = jnp.dot(rin, w1t_ref[...], preferred_element_type=jnp.float32) + b1_ref[...]
    a = jnp.maximum(pre, 0.0)
    logits = jnp.dot(a, w2t_ref[...], preferred_element_type=jnp.float32) + b2_ref[...]
    prior = -(lnn - cen_ref[...]) ** 2
    logits = (1.0 - 0.35) * logits + 0.35 * prior
    m = jnp.max(logits, axis=1, keepdims=True)
    ex = jnp.exp(logits - m)
    t4 = ex[:, 0:4] + ex[:, 4:8]
    t2 = t4[:, 0:2] + t4[:, 2:4]
    probs = ex / (t2[:, 0:1] + t2[:, 1:2])
    i8 = lax.broadcasted_iota(jnp.int32, (1, NEXP), 1)
    m1 = jnp.max(probs, axis=1, keepdims=True)
    idx1 = jnp.min(jnp.where(probs == m1, i8, 99), axis=1, keepdims=True)
    mask1 = (i8 == idx1)
    pno1 = jnp.where(mask1, -1.0, probs)
    m2 = jnp.max(pno1, axis=1, keepdims=True)
    idx2 = jnp.min(jnp.where(pno1 == m2, i8, 99), axis=1, keepdims=True)
    mask2 = (i8 == idx2)
    denom = m1 + m2 + 1e-08
    out_ref[...] = (mask1 * (m1 / denom) + mask2 * (m2 / denom)).astype(jnp.float32)


def _router(hcat, batch_col, gfnt, lnn_row, cen, w1t, b1r, w2t, b2r):
    return pl.pallas_call(
        _router_body,
        grid=(NBLK,),
        in_specs=[
            pl.BlockSpec((2, RB, HALF), lambda i: (0, i, 0)),
            pl.BlockSpec((RB, 1), lambda i: (i, 0)),
            pl.BlockSpec((3, NG), lambda i: (0, 0)),
            pl.BlockSpec((1, NG), lambda i: (0, 0)),
            pl.BlockSpec((1, NEXP), lambda i: (0, 0)),
            pl.BlockSpec((HID + 3, RH), lambda i: (0, 0)),
            pl.BlockSpec((1, RH), lambda i: (0, 0)),
            pl.BlockSpec((RH, NEXP), lambda i: (0, 0)),
            pl.BlockSpec((1, NEXP), lambda i: (0, 0)),
        ],
        out_specs=pl.BlockSpec((RB, NEXP), lambda i: (i, 0)),
        out_shape=jax.ShapeDtypeStruct((N, NEXP), jnp.float32),
    )(hcat, batch_col, gfnt, lnn_row, cen, w1t, b1r, w2t, b2r)


def _gconv_body(agg_ref, x_ref, wrel_ref, wroot_ref, b_ref, out_ref):
    agg = jnp.concatenate([agg_ref[0, 0], agg_ref[0, 1]], axis=1)
    xx = jnp.concatenate([x_ref[0, 0], x_ref[0, 1]], axis=1)
    dn = (((1,), (1,)), ((), ()))
    z = (lax.dot_general(agg, wrel_ref[0], dn, preferred_element_type=jnp.float32)
         + lax.dot_general(xx, wroot_ref[0], dn, preferred_element_type=jnp.float32)
         + b_ref[0])
    z = jnp.maximum(z, 0.0)
    out_ref[0, 0] = z[:, :HALF]
    out_ref[0, 1] = z[:, HALF:]


def _gconv(agg4, x4, wrel, wroot, b3d):
    ka = agg4.shape[0]
    kx = x4.shape[0]
    return pl.pallas_call(
        _gconv_body,
        grid=(NEXP, NBLK),
        in_specs=[
            pl.BlockSpec((1, 2, RB, HALF),
                         lambda e, i, ka=ka: (jnp.minimum(e, ka - 1), 0, i, 0)),
            pl.BlockSpec((1, 2, RB, HALF),
                         lambda e, i, kx=kx: (jnp.minimum(e, kx - 1), 0, i, 0)),
            pl.BlockSpec((1, HID, HID), lambda e, i: (e, 0, 0)),
            pl.BlockSpec((1, HID, HID), lambda e, i: (e, 0, 0)),
            pl.BlockSpec((1, 1, HID), lambda e, i: (e, 0, 0)),
        ],
        out_specs=pl.BlockSpec((1, 2, RB, HALF), lambda e, i: (e, 0, i, 0)),
        out_shape=jax.ShapeDtypeStruct((NEXP, 2, N, HALF), jnp.float32),
    )(agg4, x4, wrel, wroot, b3d)


def _gconv2_body(agg_ref, x_ref, wrel_ref, wroot_ref, b_ref, wrel3_ref,
                 out_ref, m_ref):
    agg = jnp.concatenate([agg_ref[0, 0], agg_ref[0, 1]], axis=1)
    xx = jnp.concatenate([x_ref[0, 0], x_ref[0, 1]], axis=1)
    dn = (((1,), (1,)), ((), ()))
    z = (lax.dot_general(agg, wrel_ref[0], dn, preferred_element_type=jnp.float32)
         + lax.dot_general(xx, wroot_ref[0], dn, preferred_element_type=jnp.float32)
         + b_ref[0])
    z = jnp.maximum(z, 0.0)
    out_ref[0, 0] = z[:, :HALF]
    out_ref[0, 1] = z[:, HALF:]
    m_ref[0] = lax.dot_general(z, wrel3_ref[0], dn,
                               preferred_element_type=jnp.float32)


def _gconv2(agg4, x4, wrel, wroot, b3d, wrel3):
    ne = wrel.shape[0]
    return pl.pallas_call(
        _gconv2_body,
        grid=(ne, NBLK),
        in_specs=[
            pl.BlockSpec((1, 2, RB, HALF), lambda e, i: (e, 0, i, 0)),
            pl.BlockSpec((1, 2, RB, HALF), lambda e, i: (e, 0, i, 0)),
            pl.BlockSpec((1, HID, HID), lambda e, i: (e, 0, 0)),
            pl.BlockSpec((1, HID, HID), lambda e, i: (e, 0, 0)),
            pl.BlockSpec((1, 1, HID), lambda e, i: (e, 0, 0)),
            pl.BlockSpec((1, OUT, HID), lambda e, i: (e, 0, 0)),
        ],
        out_specs=[
            pl.BlockSpec((1, 2, RB, HALF), lambda e, i: (e, 0, i, 0)),
            pl.BlockSpec((1, RB, OUT), lambda e, i: (e, i, 0)),
        ],
        out_shape=[
            jax.ShapeDtypeStruct((ne, 2, N, HALF), jnp.float32),
            jax.ShapeDtypeStruct((ne, N, OUT), jnp.float32),
        ],
    )(agg4, x4, wrel, wroot, b3d, wrel3)


def _make_l3_body(ebase, with_init):
    def body(*refs):
        if with_init:
            agg_ref, x_ref, wroot_ref, b_ref, w_ref, init_ref, out_ref = refs
        else:
            agg_ref, x_ref, wroot_ref, b_ref, w_ref = refs[:5]
            out_ref = refs[5]
        e = pl.program_id(1)
        xx = jnp.concatenate([x_ref[0, 0], x_ref[0, 1]], axis=1)
        dn = (((1,), (1,)), ((), ()))
        o = (agg_ref[0]
             + lax.dot_general(xx, wroot_ref[0], dn,
                               preferred_element_type=jnp.float32)
             + b_ref[0])
        i8 = lax.broadcasted_iota(jnp.int32, (1, NEXP), 1)
        wcol = jnp.sum(w_ref[...] * (i8 == e + ebase).astype(jnp.float32),
                       axis=1, keepdims=True)

        @pl.when(e == 0)
        def _():
            if with_init:
                out_ref[...] = init_ref[...] + o * wcol
            else:
                out_ref[...] = o * wcol

        @pl.when(e > 0)
        def _():
            out_ref[...] += o * wcol

    return body


def _l3(agg3, x4, wroot, b3d, wts, ebase, init=None):
    ne = wroot.shape[0]
    in_specs = [
        pl.BlockSpec((1, RB, OUT), lambda i, e: (e, i, 0)),
        pl.BlockSpec((1, 2, RB, HALF), lambda i, e: (e, 0, i, 0)),
        pl.BlockSpec((1, OUT, HID), lambda i, e: (e, 0, 0)),
        pl.BlockSpec((1, 1, OUT), lambda i, e: (e, 0, 0)),
        pl.BlockSpec((RB, NEXP), lambda i, e: (i, 0)),
    ]
    args = [agg3, x4, wroot, b3d, wts]
    if init is not None:
        in_specs.append(pl.BlockSpec((RB, OUT), lambda i, e: (i, 0)))
        args.append(init)
    return pl.pallas_call(
        _make_l3_body(ebase, init is not None),
        grid=(NBLK, ne),
        in_specs=in_specs,
        out_specs=pl.BlockSpec((RB, OUT), lambda i, e: (i, 0)),
        out_shape=jax.ShapeDtypeStruct((N, OUT), jnp.float32),
    )(*args)


def kernel(x, edge_index, batch, W_enc1, b_enc1, W_enc2, b_enc2, W_r1, b_r1,
           W_r2, b_r2, size_centers, Wrel1, Wroot1, b1, Wrel2, Wroot2, b2,
           Wrel3, Wroot3, b3):
    src = edge_index[0]
    dst = edge_index[1]

    pad = EP - E
    padrows = (jnp.arange(pad, dtype=jnp.int32) % 16)
    srcp = jnp.concatenate([src, padrows])
    dstp = jnp.concatenate([dst, N + padrows])
    zrows = jnp.zeros((ZR, HALF), jnp.float32)

    batch_row = batch.reshape(1, N)
    batch_col = batch.reshape(N, 1)
    src3 = src.reshape(NEB, 1, EB)

    w1t = W_enc1.T
    b1r = b_enc1.reshape(1, HID)
    w2t = W_enc2.T
    b2r = b_enc2.reshape(1, HID)
    w1rt = W_r1.T
    br1 = b_r1.reshape(1, RH)
    w2rt = W_r2.T
    br2 = b_r2.reshape(1, NEXP)
    cen = size_centers.reshape(1, NEXP)
    b1_3 = b1.reshape(NEXP, 1, HID)
    b2_3 = b2.reshape(NEXP, 1, HID)
    b3_3 = b3.reshape(NEXP, 1, OUT)

    hcat = _enc(x, w1t, b1r, w2t, b2r)
    nraw, f = _stats(batch_row, src3)
    gfn, lnn_g = _pergraph(nraw, f)
    wts = _router(hcat, batch_col, gfn.T, lnn_g.T, cen, w1rt, br1, w2rt, br2)

    agg_h = _seg1(hcat.reshape(2 * N, HALF), srcp, dstp, zrows)
    h1 = _gconv(agg_h.reshape(1, 2, N, HALF), hcat.reshape(1, 2, N, HALF),
                Wrel1, Wroot1, b1_3)
    agg1 = _seg8(h1.reshape(2 * NEXP * N, HALF), srcp, dstp,
                 zrows).reshape(NEXP, 2, N, HALF)
    h2, m2 = _gconv2(agg1, h1, Wrel2, Wroot2, b2_3, Wrel3)
    agg3 = _segm(m2.reshape(NEXP * N, HALF), srcp, dstp,
                 zrows).reshape(NEXP, N, OUT)
    pred = _l3(agg3, h2, Wroot3, b3_3, wts, 0)
    return pred


# speedup vs baseline: 1.2017x; 1.1950x over previous
import functools

import jax
import jax.numpy as jnp
from jax import lax
from jax.experimental import pallas as pl
from jax.experimental.pallas import tpu as pltpu
from jax.experimental.pallas import tpu_sc as plsc

N = 10000
E = 320000
IN = 128
HID = 256
OUT = 128
NEXP = 8
NG = 64
RH = 128
HALF = 128

NCORES = 2
NTILES = 16

W = 128
EP = 157 * NTILES * W
WPT = EP // (NTILES * W)
NPAD = N + 16
ZR = 632
ZR_LAST = NPAD - 15 * ZR
ORT = 632
ORT_LAST = N - 15 * ORT

RB = 2000
NBLK = N // RB
EB = 20000
NEB = E // EB


def _make_seg_body(ke, off_fn):
    NS = 3

    def body(x, src, dst, zrows, out, svs, dvs, rvs, gsems, ssems, svsems,
             dvsems, acc):
        c = lax.axis_index("c")
        s = lax.axis_index("s")
        ebase = s * (WPT * W)

        for e in range(ke):
            roff = off_fn(c, e)

            @pl.when(s < NTILES - 1)
            def _():
                pltpu.sync_copy(zrows, acc.at[pl.ds(s * ZR, ZR)])

            @pl.when(s == NTILES - 1)
            def _():
                pltpu.sync_copy(zrows.at[pl.ds(0, ZR_LAST)],
                                acc.at[pl.ds(15 * ZR, ZR_LAST)])

            plsc.subcore_barrier()

            def launch(w, b, drain_scatter):
                if drain_scatter:
                    pltpu.make_async_copy(rvs[b], acc.at[dvs[b]],
                                          ssems[b]).wait()
                off = ebase + w * W
                pltpu.async_copy(dst.at[pl.ds(off, W)], dvs[b], dvsems[b])
                pltpu.make_async_copy(src.at[pl.ds(off, W)], svs[b],
                                      svsems[b]).wait()
                for j in range(W // 16):
                    svs[b][pl.ds(j * 16, 16)] = (svs[b][pl.ds(j * 16, 16)]
                                                 + roff)
                pltpu.async_copy(x.at[svs[b]], rvs[b], gsems[b])

            for b in range(NS):
                pltpu.async_copy(src.at[pl.ds(ebase + b * W, W)], svs[b],
                                 svsems[b])
            for b in range(2):
                launch(b, b, False)

            def visit(w, carry):
                for b in range(NS):
                    @pl.when(w % NS == b)
                    def _(b=b):
                        pltpu.make_async_copy(x.at[svs[b]], rvs[b],
                                              gsems[b]).wait()
                        pltpu.make_async_copy(dst.at[pl.ds(0, W)], dvs[b],
                                              dvsems[b]).wait()
                        pltpu.async_copy(rvs[b], acc.at[dvs[b]], ssems[b],
                                         add=True)

                        @pl.when(w + 3 < WPT)
                        def _():
                            pltpu.async_copy(
                                src.at[pl.ds(ebase + (w + 3) * W, W)],
                                svs[b], svsems[b])
                @pl.when(w + 2 < WPT)
                def _():
                    for b in range(NS):
                        @pl.when((w + 2) % NS == b)
                        def _(b=b):
                            @pl.when(w >= 1)
                            def _():
                                pltpu.make_async_copy(
                                    rvs[b], acc.at[dvs[b]], ssems[b]).wait()
                            off = ebase + (w + 2) * W
                            pltpu.async_copy(dst.at[pl.ds(off, W)], dvs[b],
                                             dvsems[b])
                            pltpu.make_async_copy(
                                src.at[pl.ds(off, W)], svs[b],
                                svsems[b]).wait()
                            for j in range(W // 16):
                                svs[b][pl.ds(j * 16, 16)] = (
                                    svs[b][pl.ds(j * 16, 16)] + roff)
                            pltpu.async_copy(x.at[svs[b]], rvs[b], gsems[b])
                return carry

            lax.fori_loop(0, WPT, visit, 0)
            for wlast in range(WPT - 3, WPT):
                b = wlast % NS
                pltpu.make_async_copy(rvs[b], acc.at[dvs[b]], ssems[b]).wait()
            plsc.subcore_barrier()

            @pl.when(s < NTILES - 1)
            def _():
                pltpu.sync_copy(acc.at[pl.ds(s * ORT, ORT)],
                                out.at[pl.ds(roff + s * ORT, ORT)])

            @pl.when(s == NTILES - 1)
            def _():
                pltpu.sync_copy(acc.at[pl.ds(15 * ORT, ORT_LAST)],
                                out.at[pl.ds(roff + 15 * ORT, ORT_LAST)])

    return body


@functools.cache
def _seg_sc(n_rows, ke, kind):
    if kind == "halves":
        off_fn = lambda c, e: (e * NCORES + c) * N
    else:
        off_fn = lambda c, e: (c * ke + e) * N
    return pl.kernel(
        _make_seg_body(ke, off_fn),
        out_type=jax.ShapeDtypeStruct((n_rows, HALF), jnp.float32),
        mesh=plsc.VectorSubcoreMesh(core_axis_name="c", subcore_axis_name="s",
                                    num_cores=NCORES, num_subcores=NTILES),
        scratch_types=[
            [pltpu.VMEM((W,), jnp.int32) for _ in range(3)],
            [pltpu.VMEM((W,), jnp.int32) for _ in range(3)],
            [pltpu.VMEM((W, HALF), jnp.float32) for _ in range(3)],
            [pltpu.SemaphoreType.DMA for _ in range(3)],
            [pltpu.SemaphoreType.DMA for _ in range(3)],
            [pltpu.SemaphoreType.DMA for _ in range(3)],
            [pltpu.SemaphoreType.DMA for _ in range(3)],
            pltpu.VMEM_SHARED((NPAD, HALF), jnp.float32),
        ],
    )


def _seg1(xflat, srcp, dstp, zrows):
    return _seg_sc(2 * N, 1, "halves")(xflat, srcp, dstp, zrows)


def _seg8(xflat, srcp, dstp, zrows):
    return _seg_sc(2 * NEXP * N, NEXP, "halves")(xflat, srcp, dstp, zrows)


def _segm(xflat, srcp, dstp, zrows):
    return _seg_sc(NEXP * N, NEXP // NCORES, "whole")(xflat, srcp, dstp, zrows)


def _enc_body(x_ref, w1t_ref, b1_ref, w2t_ref, b2_ref, out_ref):
    h1 = jnp.maximum(
        jnp.dot(x_ref[...], w1t_ref[...], preferred_element_type=jnp.float32)
        + b1_ref[...], 0.0)
    hh = jnp.dot(h1, w2t_ref[...], preferred_element_type=jnp.float32) + b2_ref[...]
    out_ref[0] = hh[:, :HALF]
    out_ref[1] = hh[:, HALF:]


def _enc(x, w1t, b1r, w2t, b2r):
    return pl.pallas_call(
        _enc_body,
        grid=(NBLK,),
        in_specs=[
            pl.BlockSpec((RB, IN), lambda i: (i, 0)),
            pl.BlockSpec((IN, HID), lambda i: (0, 0)),
            pl.BlockSpec((1, HID), lambda i: (0, 0)),
            pl.BlockSpec((HID, HID), lambda i: (0, 0)),
            pl.BlockSpec((1, HID), lambda i: (0, 0)),
        ],
        out_specs=pl.BlockSpec((2, RB, HALF), lambda i: (0, i, 0)),
        out_shape=jax.ShapeDtypeStruct((2, N, HALF), jnp.float32),
    )(x, w1t, b1r, w2t, b2r)


def _stats_body(batch_ref, src_ref, nraw_ref, f_ref, starts_ref):
    i = pl.program_id(0)

    @pl.when(i == 0)
    def _():
        b = batch_ref[...]
        g = lax.broadcasted_iota(jnp.int32, (NG, 1), 0)
        nr = jnp.sum((b == g).astype(jnp.float32), axis=1, keepdims=True)
        nraw_ref[...] = nr
        lt = (lax.broadcasted_iota(jnp.int32, (NG + 1, NG), 1)
              < lax.broadcasted_iota(jnp.int32, (NG + 1, NG), 0)).astype(jnp.float32)
        starts_ref[...] = jnp.dot(lt, nr, preferred_element_type=jnp.float32)
        f_ref[...] = jnp.zeros((NG + 1, 1), jnp.float32)

    @pl.when(i > 0)
    def _():
        s = src_ref[0].astype(jnp.float32)
        st = starts_ref[...]
        f_ref[...] += jnp.sum((s < st).astype(jnp.float32), axis=1, keepdims=True)


def _stats(batch_row, src3):
    return pl.pallas_call(
        _stats_body,
        grid=(NEB + 1,),
        in_specs=[
            pl.BlockSpec((1, N), lambda i: (0, 0)),
            pl.BlockSpec((1, 1, EB), lambda i: (jnp.maximum(i - 1, 0), 0, 0)),
        ],
        out_specs=[
            pl.BlockSpec((NG, 1), lambda i: (0, 0)),
            pl.BlockSpec((NG + 1, 1), lambda i: (0, 0)),
        ],
        out_shape=[
            jax.ShapeDtypeStruct((NG, 1), jnp.float32),
            jax.ShapeDtypeStruct((NG + 1, 1), jnp.float32),
        ],
        scratch_shapes=[pltpu.VMEM((NG + 1, 1), jnp.float32)],
    )(batch_row, src3)


def _pergraph_body(nraw_ref, f_ref, gfn_ref, lnn_ref):
    nr = nraw_ref[...]
    f = f_ref[...]
    e = f[1:NG + 1] - f[0:NG]
    n = jnp.maximum(nr, 1.0)
    dens = e / jnp.maximum(n * (n - 1.0), 1.0)
    ln = jnp.log(n)
    lnn = (ln - jnp.min(ln)) / (jnp.max(ln) - jnp.min(ln) + 1e-06)
    gf = jnp.concatenate([n, e, dens], axis=1)
    mu = jnp.mean(gf, axis=0, keepdims=True)
    sd = jnp.sqrt(jnp.mean((gf - mu) ** 2, axis=0, keepdims=True))
    gfn_ref[...] = (gf - mu) / (sd + 1e-06)
    lnn_ref[...] = lnn


def _pergraph(nraw, f):
    return pl.pallas_call(
        _pergraph_body,
        out_shape=[
            jax.ShapeDtypeStruct((NG, 3), jnp.float32),
            jax.ShapeDtypeStruct((NG, 1), jnp.float32),
        ],
    )(nraw, f)


def _router_body(h_ref, batch_ref, gfnt_ref, lnn_ref, cen_ref, w1t_ref, b1_ref,
                 w2t_ref, b2_ref, out_ref):
    h = jnp.concatenate([h_ref[0], h_ref[1]], axis=1)
    g = lax.broadcasted_iota(jnp.int32, (1, NG), 1)
    oh = batch_ref[...] == g
    def sel(row):
        return jnp.sum(jnp.where(oh, row, 0.0), axis=1, keepdims=True)
    sf = jnp.concatenate([sel(gfnt_ref[0:1]), sel(gfnt_ref[1:2]),
                          sel(gfnt_ref[2:3])], axis=1)
    lnn = sel(lnn_ref[...])
    rin = jnp.concatenate([h, sf], axis=1)
    pre = jnp.dot(rin, w1t_ref[...], preferred_element_type=jnp.float32) + b1_ref[...]
    a = jnp.maximum(pre, 0.0)
    logits = jnp.dot(a, w2t_ref[...], preferred_element_type=jnp.float32) + b2_ref[...]
    prior = -(lnn - cen_ref[...]) ** 2
    logits = (1.0 - 0.35) * logits + 0.35 * prior
    m = jnp.max(logits, axis=1, keepdims=True)
    ex = jnp.exp(logits - m)
    t4 = ex[:, 0:4] + ex[:, 4:8]
    t2 = t4[:, 0:2] + t4[:, 2:4]
    probs = ex / (t2[:, 0:1] + t2[:, 1:2])
    i8 = lax.broadcasted_iota(jnp.int32, (1, NEXP), 1)
    m1 = jnp.max(probs, axis=1, keepdims=True)
    idx1 = jnp.min(jnp.where(probs == m1, i8, 99), axis=1, keepdims=True)
    mask1 = (i8 == idx1)
    pno1 = jnp.where(mask1, -1.0, probs)
    m2 = jnp.max(pno1, axis=1, keepdims=True)
    idx2 = jnp.min(jnp.where(pno1 == m2, i8, 99), axis=1, keepdims=True)
    mask2 = (i8 == idx2)
    denom = m1 + m2 + 1e-08
    out_ref[...] = (mask1 * (m1 / denom) + mask2 * (m2 / denom)).astype(jnp.float32)


def _router(hcat, batch_col, gfnt, lnn_row, cen, w1t, b1r, w2t, b2r):
    return pl.pallas_call(
        _router_body,
        grid=(NBLK,),
        in_specs=[
            pl.BlockSpec((2, RB, HALF), lambda i: (0, i, 0)),
            pl.BlockSpec((RB, 1), lambda i: (i, 0)),
            pl.BlockSpec((3, NG), lambda i: (0, 0)),
            pl.BlockSpec((1, NG), lambda i: (0, 0)),
            pl.BlockSpec((1, NEXP), lambda i: (0, 0)),
            pl.BlockSpec((HID + 3, RH), lambda i: (0, 0)),
            pl.BlockSpec((1, RH), lambda i: (0, 0)),
            pl.BlockSpec((RH, NEXP), lambda i: (0, 0)),
            pl.BlockSpec((1, NEXP), lambda i: (0, 0)),
        ],
        out_specs=pl.BlockSpec((RB, NEXP), lambda i: (i, 0)),
        out_shape=jax.ShapeDtypeStruct((N, NEXP), jnp.float32),
    )(hcat, batch_col, gfnt, lnn_row, cen, w1t, b1r, w2t, b2r)


def _gconv_body(agg_ref, x_ref, wrel_ref, wroot_ref, b_ref, out_ref):
    agg = jnp.concatenate([agg_ref[0, 0], agg_ref[0, 1]], axis=1)
    xx = jnp.concatenate([x_ref[0, 0], x_ref[0, 1]], axis=1)
    dn = (((1,), (1,)), ((), ()))
    z = (lax.dot_general(agg, wrel_ref[0], dn, preferred_element_type=jnp.float32)
         + lax.dot_general(xx, wroot_ref[0], dn, preferred_element_type=jnp.float32)
         + b_ref[0])
    z = jnp.maximum(z, 0.0)
    out_ref[0, 0] = z[:, :HALF]
    out_ref[0, 1] = z[:, HALF:]


def _gconv(agg4, x4, wrel, wroot, b3d):
    ka = agg4.shape[0]
    kx = x4.shape[0]
    return pl.pallas_call(
        _gconv_body,
        grid=(NEXP, NBLK),
        in_specs=[
            pl.BlockSpec((1, 2, RB, HALF),
                         lambda e, i, ka=ka: (jnp.minimum(e, ka - 1), 0, i, 0)),
            pl.BlockSpec((1, 2, RB, HALF),
                         lambda e, i, kx=kx: (jnp.minimum(e, kx - 1), 0, i, 0)),
            pl.BlockSpec((1, HID, HID), lambda e, i: (e, 0, 0)),
            pl.BlockSpec((1, HID, HID), lambda e, i: (e, 0, 0)),
            pl.BlockSpec((1, 1, HID), lambda e, i: (e, 0, 0)),
        ],
        out_specs=pl.BlockSpec((1, 2, RB, HALF), lambda e, i: (e, 0, i, 0)),
        out_shape=jax.ShapeDtypeStruct((NEXP, 2, N, HALF), jnp.float32),
    )(agg4, x4, wrel, wroot, b3d)


def _gconv2_body(agg_ref, x_ref, wrel_ref, wroot_ref, b_ref, wrel3_ref,
                 out_ref, m_ref):
    agg = jnp.concatenate([agg_ref[0, 0], agg_ref[0, 1]], axis=1)
    xx = jnp.concatenate([x_ref[0, 0], x_ref[0, 1]], axis=1)
    dn = (((1,), (1,)), ((), ()))
    z = (lax.dot_general(agg, wrel_ref[0], dn, preferred_element_type=jnp.float32)
         + lax.dot_general(xx, wroot_ref[0], dn, preferred_element_type=jnp.float32)
         + b_ref[0])
    z = jnp.maximum(z, 0.0)
    out_ref[0, 0] = z[:, :HALF]
    out_ref[0, 1] = z[:, HALF:]
    m_ref[0] = lax.dot_general(z, wrel3_ref[0], dn,
                               preferred_element_type=jnp.float32)


def _gconv2(agg4, x4, wrel, wroot, b3d, wrel3):
    ne = wrel.shape[0]
    return pl.pallas_call(
        _gconv2_body,
        grid=(ne, NBLK),
        in_specs=[
            pl.BlockSpec((1, 2, RB, HALF), lambda e, i: (e, 0, i, 0)),
            pl.BlockSpec((1, 2, RB, HALF), lambda e, i: (e, 0, i, 0)),
            pl.BlockSpec((1, HID, HID), lambda e, i: (e, 0, 0)),
            pl.BlockSpec((1, HID, HID), lambda e, i: (e, 0, 0)),
            pl.BlockSpec((1, 1, HID), lambda e, i: (e, 0, 0)),
            pl.BlockSpec((1, OUT, HID), lambda e, i: (e, 0, 0)),
        ],
        out_specs=[
            pl.BlockSpec((1, 2, RB, HALF), lambda e, i: (e, 0, i, 0)),
            pl.BlockSpec((1, RB, OUT), lambda e, i: (e, i, 0)),
        ],
        out_shape=[
            jax.ShapeDtypeStruct((ne, 2, N, HALF), jnp.float32),
            jax.ShapeDtypeStruct((ne, N, OUT), jnp.float32),
        ],
    )(agg4, x4, wrel, wroot, b3d, wrel3)


def _make_l3_body(ebase, with_init):
    def body(*refs):
        if with_init:
            agg_ref, x_ref, wroot_ref, b_ref, w_ref, init_ref, out_ref = refs
        else:
            agg_ref, x_ref, wroot_ref, b_ref, w_ref = refs[:5]
            out_ref = refs[5]
        e = pl.program_id(1)
        xx = jnp.concatenate([x_ref[0, 0], x_ref[0, 1]], axis=1)
        dn = (((1,), (1,)), ((), ()))
        o = (agg_ref[0]
             + lax.dot_general(xx, wroot_ref[0], dn,
                               preferred_element_type=jnp.float32)
             + b_ref[0])
        i8 = lax.broadcasted_iota(jnp.int32, (1, NEXP), 1)
        wcol = jnp.sum(w_ref[...] * (i8 == e + ebase).astype(jnp.float32),
                       axis=1, keepdims=True)

        @pl.when(e == 0)
        def _():
            if with_init:
                out_ref[...] = init_ref[...] + o * wcol
            else:
                out_ref[...] = o * wcol

        @pl.when(e > 0)
        def _():
            out_ref[...] += o * wcol

    return body


def _l3(agg3, x4, wroot, b3d, wts, ebase, init=None):
    ne = wroot.shape[0]
    in_specs = [
        pl.BlockSpec((1, RB, OUT), lambda i, e: (e, i, 0)),
        pl.BlockSpec((1, 2, RB, HALF), lambda i, e: (e, 0, i, 0)),
        pl.BlockSpec((1, OUT, HID), lambda i, e: (e, 0, 0)),
        pl.BlockSpec((1, 1, OUT), lambda i, e: (e, 0, 0)),
        pl.BlockSpec((RB, NEXP), lambda i, e: (i, 0)),
    ]
    args = [agg3, x4, wroot, b3d, wts]
    if init is not None:
        in_specs.append(pl.BlockSpec((RB, OUT), lambda i, e: (i, 0)))
        args.append(init)
    return pl.pallas_call(
        _make_l3_body(ebase, init is not None),
        grid=(NBLK, ne),
        in_specs=in_specs,
        out_specs=pl.BlockSpec((RB, OUT), lambda i, e: (i, 0)),
        out_shape=jax.ShapeDtypeStruct((N, OUT), jnp.float32),
    )(*args)


def kernel(x, edge_index, batch, W_enc1, b_enc1, W_enc2, b_enc2, W_r1, b_r1,
           W_r2, b_r2, size_centers, Wrel1, Wroot1, b1, Wrel2, Wroot2, b2,
           Wrel3, Wroot3, b3):
    src = edge_index[0]
    dst = edge_index[1]

    pad = EP - E
    padrows = (jnp.arange(pad, dtype=jnp.int32) % 16)
    srcp = jnp.concatenate([src, padrows])
    dstp = jnp.concatenate([dst, N + padrows])
    zrows = jnp.zeros((ZR, HALF), jnp.float32)

    batch_row = batch.reshape(1, N)
    batch_col = batch.reshape(N, 1)
    src3 = src.reshape(NEB, 1, EB)

    w1t = W_enc1.T
    b1r = b_enc1.reshape(1, HID)
    w2t = W_enc2.T
    b2r = b_enc2.reshape(1, HID)
    w1rt = W_r1.T
    br1 = b_r1.reshape(1, RH)
    w2rt = W_r2.T
    br2 = b_r2.reshape(1, NEXP)
    cen = size_centers.reshape(1, NEXP)
    b1_3 = b1.reshape(NEXP, 1, HID)
    b2_3 = b2.reshape(NEXP, 1, HID)
    b3_3 = b3.reshape(NEXP, 1, OUT)

    hcat = _enc(x, w1t, b1r, w2t, b2r)
    nraw, f = _stats(batch_row, src3)
    gfn, lnn_g = _pergraph(nraw, f)
    wts = _router(hcat, batch_col, gfn.T, lnn_g.T, cen, w1rt, br1, w2rt, br2)

    agg_h = _seg1(hcat.reshape(2 * N, HALF), srcp, dstp, zrows)
    h1 = _gconv(agg_h.reshape(1, 2, N, HALF), hcat.reshape(1, 2, N, HALF),
                Wrel1, Wroot1, b1_3)
    agg1 = _seg8(h1.reshape(2 * NEXP * N, HALF), srcp, dstp,
                 zrows).reshape(NEXP, 2, N, HALF)
    h2, m2 = _gconv2(agg1, h1, Wrel2, Wroot2, b2_3, Wrel3)
    agg3 = _segm(m2.reshape(NEXP * N, HALF), srcp, dstp,
                 zrows).reshape(NEXP, N, OUT)
    pred = _l3(agg3, h2, Wroot3, b3_3, wts, 0)
    return pred
